# baseline (speedup 1.0000x reference)
import functools

import jax
import jax.numpy as jnp
from jax.experimental import pallas as pl


def _combine_body(o0, o1, o2, o3, sc, out_ref):
    acc = o0[...] * sc[:, 0:1]
    acc += o1[...] * sc[:, 1:2]
    acc += o2[...] * sc[:, 2:3]
    acc += o3[...] * sc[:, 3:4]
    out_ref[...] = acc


def _combine(outs, sc_e):
    E, D = outs[0].shape
    BLK = 4000
    grid = (E // BLK,)
    spec = pl.BlockSpec((BLK, D), lambda i: (i, 0))
    spec_sc = pl.BlockSpec((BLK, 4), lambda i: (i, 0))
    return pl.pallas_call(
        _combine_body,
        grid=grid,
        in_specs=[spec, spec, spec, spec, spec_sc],
        out_specs=spec,
        out_shape=jax.ShapeDtypeStruct((E, D), jnp.float32),
    )(*outs, sc_e)


def kernel(x, edge_index, edge_attr, edge_index_bond, edge_index_batch,
           W_u, W_v, W_e, W_rel, b_rel, W_root, a, W_gout, b_gout, a_bias):
    E_loc = edge_attr.shape[0]
    N = x.shape[0]
    B = 128
    src, dst = edge_index_bond[0], edge_index_bond[1]
    seg = edge_index_batch

    edge_u = x @ W_u
    edge_v = x @ W_v
    edge_uv = edge_attr @ W_e
    ea = (edge_u[edge_index[0]] + edge_v[edge_index[1]] + edge_uv) / 3.0

    w_rel = W_rel[:, 0]
    w_root = W_root[:, 0]

    outs = []
    out = ea
    for n in range(4):
        agg = jax.ops.segment_sum(out[src], dst, num_segments=E_loc)
        out = ea + agg
        outs.append(out)

    rs = [o @ w_rel for o in outs]
    ts = [o @ w_root for o in outs]
    us = [jax.ops.segment_sum(r[src], dst, num_segments=E_loc) for r in rs]

    gxs = []
    for n in range(4):
        x_conv = us[n] + b_rel[0] + ts[n]
        m = jax.ops.segment_max(x_conv, seg, num_segments=B)
        x_exp = jnp.exp(x_conv - m[seg])
        denom = jax.ops.segment_sum(x_exp, seg, num_segments=B)
        scores = x_exp / (denom[seg] + 1e-16)
        gx = jax.ops.segment_sum(outs[n] * scores[:, None], seg, num_segments=B)
        gxs.append(gx)

    gout_all = jnp.stack([jnp.tanh(g @ W_gout + b_gout) for g in gxs], axis=-1)
    sc = jnp.sum(gout_all * a, axis=1, keepdims=True) + a_bias
    sc = jax.nn.softmax(sc, axis=-1)
    sc_e = sc[seg, 0, :]

    out_final = _combine(outs, sc_e)
    x_new = x + jax.ops.segment_sum(out_final, edge_index[1], num_segments=N)
    return x_new


# speedup vs baseline: 1.7182x; 1.7182x over previous
import functools

import jax
import jax.numpy as jnp
from jax import lax
from jax.experimental import pallas as pl
from jax.experimental.pallas import tpu as pltpu, tpu_sc as plsc

E = 320000
EB = 320000
N_NODES = 10000
D = 128
B_GRAPHS = 128
NCH = 32
CH = 10000
PER_SC = NCH // 2
B0 = 128
IOTA16 = None

_mesh = plsc.VectorSubcoreMesh(core_axis_name="c", subcore_axis_name="s",
                               num_cores=2, num_subcores=16)


_GDIMS = lax.GatherDimensionNumbers(offset_dims=(), collapsed_slice_dims=(0,),
                                    start_index_map=(0,))


def _shuffle(v, idx):
    return lax.gather(v, idx[:, None], _GDIMS, (1,), unique_indices=True,
                      mode=lax.GatherScatterMode.PROMISE_IN_BOUNDS)


def _extract(h0, h1, h2, idx):
    i16 = lax.iota(jnp.int32, 16)
    v = (jnp.where(i16 == idx, h0, 0) + jnp.where(i16 == (idx - 16), h1, 0)
         + jnp.where(i16 == (idx - 32), h2, 0))
    for s in (8, 4, 2, 1):
        v = v + _shuffle(v, i16 ^ s)
    return jnp.squeeze(lax.slice(v, (0,), (1,)))


def _matvec_body(prev_hbm, ea_hbm, srcs_hbm, dstl_hbm, bnd_hbm, out_hbm,
                 srcb, dstb, rows, wbbuf, bndv, acc, sem):
    cid = lax.axis_index("c")
    sid = lax.axis_index("s")
    pltpu.sync_copy(bnd_hbm, bndv)
    h0 = bndv[pl.ds(0, 16)]
    h1 = bndv[pl.ds(16, 16)]
    h2 = bndv[pl.ds(32, 16)]
    i16 = lax.iota(jnp.int32, 16)
    nbat = CH // 80

    def chunk_body(ci, _):
        chunk = cid * PER_SC + ci
        row0 = chunk * CH
        for kk in range(8):
            j = sid + kk * 16

            @pl.when(j < nbat)
            def _cp():
                pltpu.sync_copy(ea_hbm.at[pl.ds(row0 + j * 80, 80)], wbbuf)
                pltpu.sync_copy(wbbuf, acc.at[pl.ds(j * 80, 80)])
        plsc.subcore_barrier()

        p0 = _extract(h0, h1, h2, chunk)
        p1 = _extract(h0, h1, h2, chunk + 1)
        ln = p1 - p0
        t0 = p0 + (ln * sid) // 16
        t1 = p0 + (ln * (sid + 1)) // 16
        a0 = (t0 // B0) * B0
        nb = (t1 - a0 + (B0 - 1)) // B0

        def body(j, _):
            base = a0 + j * B0
            pltpu.sync_copy(srcs_hbm.at[pl.ds(base, B0)], srcb)
            pltpu.sync_copy(dstl_hbm.at[pl.ds(base, B0)], dstb)
            for k in range(B0 // 16):
                pos = base + k * 16 + i16
                m = (pos >= t0) & (pos < t1)
                s16 = srcb[pl.ds(k * 16, 16)]
                d16 = dstb[pl.ds(k * 16, 16)]
                srcb[pl.ds(k * 16, 16)] = jnp.where(m, s16, 0)
                dstb[pl.ds(k * 16, 16)] = jnp.where(m, d16, CH)
            pltpu.async_copy(prev_hbm.at[srcb], rows, sem).wait()
            pltpu.sync_copy(rows, acc.at[dstb], add=True)
            return 0

        lax.fori_loop(0, nb, body, 0)
        plsc.subcore_barrier()

        for kk in range(8):
            j = sid + kk * 16

            @pl.when(j < nbat)
            def _wb():
                pltpu.sync_copy(acc.at[pl.ds(j * 80, 80)], wbbuf)
                pltpu.sync_copy(wbbuf, out_hbm.at[pl.ds(row0 + j * 80, 80)])
        plsc.subcore_barrier()
        return 0

    lax.fori_loop(0, PER_SC, chunk_body, 0)


_matvec = pl.kernel(
    _matvec_body,
    out_type=jax.ShapeDtypeStruct((E, D), jnp.float32),
    mesh=_mesh,
    scratch_types=[
        pltpu.VMEM((B0,), jnp.int32),
        pltpu.VMEM((B0,), jnp.int32),
        pltpu.VMEM((B0, D), jnp.float32),
        pltpu.VMEM((80, D), jnp.float32),
        pltpu.VMEM((48,), jnp.int32),
        pltpu.VMEM_SHARED((CH + 8, D), jnp.float32),
        pltpu.SemaphoreType.DMA,
    ],
)


def _nodescatter_body(of_hbm, ei1_hbm, x_hbm, part_hbm,
                      idxb, rows, acc, sem):
    cid = lax.axis_index("c")
    sid = lax.axis_index("s")
    wid = cid * 16 + sid
    for kk in range(8):
        j = sid + kk * 16

        @pl.when(j < N_NODES // 80)
        def _cp():
            pltpu.sync_copy(x_hbm.at[pl.ds(j * 80, 80)], rows)
            pltpu.sync_copy(rows, acc.at[pl.ds(j * 80, 80)])
    plsc.subcore_barrier()

    def body(j, _):
        base = wid * (E // 32) + j * 80
        pltpu.sync_copy(ei1_hbm.at[pl.ds(base, 80)], idxb)
        pltpu.sync_copy(of_hbm.at[pl.ds(base, 80)], rows)
        pltpu.sync_copy(rows, acc.at[idxb], add=True)
        return 0

    lax.fori_loop(0, (E // 32) // 80, body, 0)
    plsc.subcore_barrier()
    for kk in range(8):
        j = sid + kk * 16

        @pl.when(j < N_NODES // 80)
        def _wb():
            pltpu.sync_copy(acc.at[pl.ds(j * 80, 80)], rows)
            pltpu.sync_copy(rows, part_hbm.at[pl.ds(cid * N_NODES + j * 80, 80)])


_nodescatter = pl.kernel(
    _nodescatter_body,
    out_type=jax.ShapeDtypeStruct((2 * N_NODES, D), jnp.float32),
    mesh=_mesh,
    scratch_types=[
        pltpu.VMEM((80,), jnp.int32),
        pltpu.VMEM((80, D), jnp.float32),
        pltpu.VMEM_SHARED((N_NODES, D), jnp.float32),
        pltpu.SemaphoreType.DMA,
    ],
)


_BLK = 2000
_NBLK = E // _BLK


def _gx_body(o0, o1, o2, o3, xe, seg, u_ref, den_ref):
    @pl.when(pl.program_id(0) == 0)
    def _init():
        u_ref[...] = jnp.zeros_like(u_ref)
        den_ref[...] = jnp.zeros_like(den_ref)

    gid = lax.broadcasted_iota(jnp.int32, (_BLK, B_GRAPHS), 1)
    eq = seg[...] == gid
    obs = (o0, o1, o2, o3)
    for n in range(4):
        w = jnp.where(eq, xe[:, n:n + 1], 0.0)
        u_ref[n] += lax.dot_general(w, obs[n][...],
                                    (((0,), (0,)), ((), ())),
                                    preferred_element_type=jnp.float32)
        den_ref[n:n + 1, :] += jnp.sum(w, axis=0, keepdims=True)


@jax.jit
def _gx_pool(outs, xe, seg2d):
    spec = pl.BlockSpec((_BLK, D), lambda i: (i, 0))
    return pl.pallas_call(
        _gx_body,
        grid=(_NBLK,),
        in_specs=[spec, spec, spec, spec,
                  pl.BlockSpec((_BLK, 4), lambda i: (i, 0)),
                  pl.BlockSpec((_BLK, 1), lambda i: (i, 0))],
        out_specs=[pl.BlockSpec((4, B_GRAPHS, D), lambda i: (0, 0, 0)),
                   pl.BlockSpec((8, B_GRAPHS), lambda i: (0, 0))],
        out_shape=[jax.ShapeDtypeStruct((4, B_GRAPHS, D), jnp.float32),
                   jax.ShapeDtypeStruct((8, B_GRAPHS), jnp.float32)],
    )(*outs, xe, seg2d)


def _combine_body(o0, o1, o2, o3, seg, sc, out_ref):
    gid = lax.broadcasted_iota(jnp.int32, (_BLK, B_GRAPHS), 1)
    oh = (seg[...] == gid).astype(jnp.float32)
    sce = oh @ sc[...]
    acc = o0[...] * sce[:, 0:1]
    acc += o1[...] * sce[:, 1:2]
    acc += o2[...] * sce[:, 2:3]
    acc += o3[...] * sce[:, 3:4]
    out_ref[...] = acc


@jax.jit
def _combine(outs, seg2d, sc):
    spec = pl.BlockSpec((_BLK, D), lambda i: (i, 0))
    return pl.pallas_call(
        _combine_body,
        grid=(_NBLK,),
        in_specs=[spec, spec, spec, spec,
                  pl.BlockSpec((_BLK, 1), lambda i: (i, 0)),
                  pl.BlockSpec((B_GRAPHS, 4), lambda i: (0, 0))],
        out_specs=spec,
        out_shape=jax.ShapeDtypeStruct((E, D), jnp.float32),
    )(*outs, seg2d, sc)


def kernel(x, edge_index, edge_attr, edge_index_bond, edge_index_batch,
           W_u, W_v, W_e, W_rel, b_rel, W_root, a, W_gout, b_gout, a_bias):
    src, dst = edge_index_bond[0], edge_index_bond[1]
    seg = edge_index_batch

    edge_u = x @ (W_u / 3.0)
    edge_v = x @ (W_v / 3.0)
    edge_uv = edge_attr @ (W_e / 3.0)
    ea = edge_u[edge_index[0]] + edge_v[edge_index[1]] + edge_uv

    perm = jnp.argsort(dst)
    dst_s = dst[perm]
    src_s = src[perm]
    dstl_s = jnp.remainder(dst_s, CH).astype(jnp.int32)
    bnd = jnp.searchsorted(dst_s, jnp.arange(NCH + 1, dtype=jnp.int32) * CH,
                           side="left").astype(jnp.int32)
    bnd = jnp.pad(bnd, (0, 48 - (NCH + 1)))

    outs = []
    prev = ea
    for n in range(4):
        prev = _matvec(prev, ea, src_s, dstl_s, bnd)
        outs.append(prev)

    w2 = jnp.concatenate([W_rel, W_root], axis=1)
    ea_rel = ea @ W_rel[:, 0]
    rts = [o @ w2 for o in outs]
    r3 = rts[3][:, 0]
    u3 = jax.ops.segment_sum(r3[src], dst, num_segments=E)
    xes = []
    ms = []
    for n in range(4):
        un = rts[n + 1][:, 0] - ea_rel if n < 3 else u3
        x_conv = un + b_rel[0] + rts[n][:, 1]
        mx = jnp.max(x_conv)
        xes.append(jnp.exp(x_conv - mx))
        ms.append(mx)
    xe = jnp.stack(xes, axis=1)

    seg2d = seg.astype(jnp.int32)[:, None]
    u, den = _gx_pool(outs, xe, seg2d)
    gx = u / (den[:4, :, None] + 1e-16)

    gout_all = jnp.stack([jnp.tanh(gx[n] @ W_gout + b_gout) for n in range(4)],
                         axis=-1)
    sc = jnp.sum(gout_all * a, axis=1, keepdims=True) + a_bias
    sc = jax.nn.softmax(sc, axis=-1)[:, 0, :]

    out_final = _combine(outs, seg2d, sc)
    part = _nodescatter(out_final, edge_index[1].astype(jnp.int32), x)
    return part[:N_NODES] + part[N_NODES:] - x
